# Initial kernel scaffold; baseline (speedup 1.0000x reference)
import jax
import jax.numpy as jnp
from jax.experimental import pallas as pl


def kernel(input, edge, W, a):
    raise NotImplementedError("write your pallas kernel here")


# speedup vs baseline: 1.4918x; 1.4918x over previous
import functools

import jax
import jax.numpy as jnp
from jax.experimental import pallas as pl

N = 10000
D = 256
ROWS_MM = 1000
ROWS_SM = 16


def _matmul_body(x_ref, w_ref, o_ref):
    o_ref[...] = jnp.dot(x_ref[...], w_ref[...],
                         preferred_element_type=jnp.float32)


def _project(x, W):
    return pl.pallas_call(
        _matmul_body,
        grid=(N // ROWS_MM,),
        in_specs=[
            pl.BlockSpec((ROWS_MM, D), lambda i: (i, 0)),
            pl.BlockSpec((D, D), lambda i: (0, 0)),
        ],
        out_specs=pl.BlockSpec((ROWS_MM, D), lambda i: (i, 0)),
        out_shape=jax.ShapeDtypeStruct((N, D), jnp.float32),
    )(x, W)


def _softmax_body(m_ref, o_ref):
    m = m_ref[...]
    mask = m != 0.0
    logits = jnp.where(mask, m, -jnp.inf)
    rowmax = jnp.max(logits, axis=1, keepdims=True)
    safe = jnp.where(jnp.isfinite(rowmax), rowmax, 0.0)
    e = jnp.where(mask, jnp.exp(m - safe), 0.0)
    denom = jnp.sum(e, axis=1, keepdims=True)
    o_ref[...] = jnp.where(denom > 0, e / jnp.where(denom > 0, denom, 1.0), 0.0)


def _row_softmax(M):
    return pl.pallas_call(
        _softmax_body,
        grid=(N // ROWS_SM,),
        in_specs=[pl.BlockSpec((ROWS_SM, N), lambda i: (i, 0))],
        out_specs=pl.BlockSpec((ROWS_SM, N), lambda i: (i, 0)),
        out_shape=jax.ShapeDtypeStruct((N, N), jnp.float32),
    )(M)


def kernel(input, edge, W, a):
    x = _project(input, W)
    src = edge[0]
    dst = edge[1]
    xs = x[src]
    xd = x[dst]
    norm = jnp.sqrt(
        jnp.sum(xs * xs, axis=1, keepdims=True)
        * jnp.sum(xd * xd, axis=1, keepdims=True))
    h = jax.nn.relu(xs * xd / norm)
    s = jnp.squeeze(h @ a)
    s = jnp.where(s > 0, s, jnp.full_like(s, -9e15))
    M = jnp.zeros((N, N), jnp.float32).at[src, dst].add(s)
    A = _row_softmax(M)
    return (x, A)
